# Initial kernel scaffold; baseline (speedup 1.0000x reference)
import jax
import jax.numpy as jnp
from jax.experimental import pallas as pl


def kernel(x):
    raise NotImplementedError("write your pallas kernel here")


# speedup vs baseline: 6.2602x; 6.2602x over previous
import functools

import jax
import jax.numpy as jnp
from jax import lax
from jax.experimental import pallas as pl
from jax.experimental.pallas import tpu as pltpu
from jax.experimental.pallas import tpu_sc as plsc

B, P, K = 16384, 1000, 4
N = B * P
NC, NS = 2, 16
NW = NC * NS
PER_W = N // NW
CHUNK = 8_000
NCHUNK = PER_W // CHUNK
OUT_CHUNK = CHUNK * K

_mesh = plsc.VectorSubcoreMesh(
    core_axis_name="c", subcore_axis_name="s", num_cores=NC, num_subcores=NS
)


@functools.partial(
    pl.kernel,
    out_type=jax.ShapeDtypeStruct((K * N,), jnp.float32),
    mesh=_mesh,
    scratch_types=[
        pltpu.VMEM((CHUNK,), jnp.float32),
        pltpu.VMEM((OUT_CHUNK,), jnp.float32),
    ],
    compiler_params=pltpu.CompilerParams(needs_layout_passes=False),
)
def _onehot_sc(x_hbm, out_hbm, x_v, out_v):
    wid = lax.axis_index("s") * NC + lax.axis_index("c")
    base = wid * PER_W

    @pl.loop(0, NCHUNK)
    def _chunk(j):
        cbase = base + j * CHUNK
        pltpu.sync_copy(x_hbm.at[pl.ds(cbase, CHUNK)], x_v)

        @pl.loop(0, CHUNK // 16)
        def _clamp(i):
            v = x_v[pl.ds(i * 16, 16)]
            x_v[pl.ds(i * 16, 16)] = jnp.clip(v, 0.0, 3.0)

        @pl.loop(0, OUT_CHUNK // 16)
        def _expand(t):
            lane = lax.iota(jnp.int32, 16)
            g = plsc.load_gather(x_v, [(lane >> 2) + t * K])
            binf = (lane & 3).astype(jnp.float32)
            out_v[pl.ds(t * 16, 16)] = jnp.where(g == binf, 1.0, 0.0)

        pltpu.sync_copy(out_v, out_hbm.at[pl.ds(cbase * K, OUT_CHUNK)])


def kernel(x):
    out = _onehot_sc(x.reshape(N))
    return out.reshape(B, P, K)


# speedup vs baseline: 205.9238x; 32.8942x over previous
import functools

import jax
import jax.numpy as jnp
from jax import lax
from jax.experimental import pallas as pl
from jax.experimental.pallas import tpu as pltpu
from jax.experimental.pallas import tpu_sc as plsc

B, P, K = 16384, 1000, 4
NC, NS = 2, 16
NW = NC * NS
NTB = B // 128
TB_PER_W = NTB // NW
PW = 200
NPW = P // PW

_mesh = plsc.VectorSubcoreMesh(
    core_axis_name="c", subcore_axis_name="s", num_cores=NC, num_subcores=NS
)


@functools.partial(
    pl.kernel,
    out_type=jax.ShapeDtypeStruct((P, NTB, K, 128), jnp.float32),
    mesh=_mesh,
    scratch_types=[
        pltpu.VMEM((PW, 128), jnp.float32),
        pltpu.VMEM((PW, K, 128), jnp.float32),
    ],
    compiler_params=pltpu.CompilerParams(needs_layout_passes=False),
)
def _onehot_sc(xt_hbm, out_hbm, x_v, out_v):
    wid = lax.axis_index("s") * NC + lax.axis_index("c")

    @pl.loop(0, TB_PER_W)
    def _tb_loop(tbi):
        tb = wid * TB_PER_W + tbi

        @pl.loop(0, NPW)
        def _pw_loop(pw):
            pltpu.sync_copy(
                xt_hbm.at[pl.ds(pw * PW, PW), pl.ds(tb * 128, 128)], x_v
            )

            @pl.loop(0, PW)
            def _p_loop(p_local):
                for blq in range(128 // 16):
                    g = x_v[p_local, pl.ds(blq * 16, 16)]
                    g = jnp.clip(g, 0.0, 3.0)
                    for k in range(K):
                        out_v[p_local, k, pl.ds(blq * 16, 16)] = jnp.where(
                            g == float(k), 1.0, 0.0
                        )

            pltpu.sync_copy(
                out_v, out_hbm.at[pl.ds(pw * PW, PW), tb, :, :]
            )


def kernel(x):
    y = _onehot_sc(x.T)
    return y.transpose(1, 3, 0, 2).reshape(B, P, K)


# speedup vs baseline: 297.4147x; 1.4443x over previous
import functools

import jax
import jax.numpy as jnp
from jax import lax
from jax.experimental import pallas as pl
from jax.experimental.pallas import tpu as pltpu
from jax.experimental.pallas import tpu_sc as plsc

B, P, K = 16384, 1000, 4
NC, NS = 2, 16
NW = NC * NS
NTB = B // 128
TB_PER_W = NTB // NW
PW = 40
NPW = P // PW
NBLK = NPW * TB_PER_W

_mesh = plsc.VectorSubcoreMesh(
    core_axis_name="c", subcore_axis_name="s", num_cores=NC, num_subcores=NS
)


@functools.partial(
    pl.kernel,
    out_type=jax.ShapeDtypeStruct((P, NTB, K, 128), jnp.float32),
    mesh=_mesh,
    scratch_types=[
        pltpu.VMEM((PW, 128), jnp.float32),
        pltpu.VMEM((PW, 128), jnp.float32),
        pltpu.VMEM((PW, K, 128), jnp.float32),
        pltpu.VMEM((PW, K, 128), jnp.float32),
        pltpu.SemaphoreType.DMA,
        pltpu.SemaphoreType.DMA,
        pltpu.SemaphoreType.DMA,
        pltpu.SemaphoreType.DMA,
    ],
)
def _onehot_sc(xt_hbm, out_hbm, x_v0, x_v1, o_v0, o_v1, si0, si1, so0, so1):
    wid = lax.axis_index("s") * NC + lax.axis_index("c")
    xv = (x_v0, x_v1)
    ov = (o_v0, o_v1)
    si = (si0, si1)
    so = (so0, so1)

    def in_window(jb):
        pw = jb >> 2
        tb = wid * TB_PER_W + (jb & 3)
        return xt_hbm.at[pl.ds(pw * PW, PW), pl.ds(tb * 128, 128)]

    def out_window(jb):
        pw = jb >> 2
        tb = wid * TB_PER_W + (jb & 3)
        return out_hbm.at[pl.ds(pw * PW, PW), tb, :, :]

    pltpu.async_copy(in_window(0), xv[0], si[0])

    @pl.loop(0, NBLK // 2)
    def _blk_loop(j):
        for b in range(2):
            jb = 2 * j + b

            @pl.when(jb + 1 < NBLK)
            def _():
                pltpu.async_copy(in_window(jb + 1), xv[1 - b], si[1 - b])

            pltpu.make_async_copy(in_window(jb), xv[b], si[b]).wait()

            @pl.when(jb >= 2)
            def _():
                pltpu.make_async_copy(ov[b], out_window(jb - 2), so[b]).wait()

            @pl.loop(0, PW)
            def _p_loop(p_local):
                for blq in range(128 // 16):
                    g = xv[b][p_local, pl.ds(blq * 16, 16)]
                    g = jnp.clip(g, 0.0, 3.0)
                    for k in range(K):
                        ov[b][p_local, k, pl.ds(blq * 16, 16)] = jnp.where(
                            g == float(k), 1.0, 0.0
                        )

            pltpu.async_copy(ov[b], out_window(jb), so[b])

    pltpu.make_async_copy(ov[0], out_window(NBLK - 2), so[0]).wait()
    pltpu.make_async_copy(ov[1], out_window(NBLK - 1), so[1]).wait()


def kernel(x):
    y = _onehot_sc(x.T)
    return y.transpose(1, 3, 0, 2).reshape(B, P, K)
